# baseline (speedup 1.0000x reference)
import functools

import jax
import jax.numpy as jnp
from jax import lax
from jax.experimental import pallas as pl
from jax.experimental.pallas import tpu as pltpu
from jax.experimental.pallas import tpu_sc as plsc

_NC = 2
_NS = 16
_NW = _NC * _NS
_LANES = 16


def _build_sc_posenc(batch, seq_len, d_model, chunk=32, nbuf=3):
    s_per_w = seq_len // _NW
    n_chunks = s_per_w // chunk
    mesh = plsc.VectorSubcoreMesh(core_axis_name="c", subcore_axis_name="s")

    @functools.partial(
        pl.kernel,
        out_type=jax.ShapeDtypeStruct((batch * seq_len, d_model), jnp.float32),
        mesh=mesh,
        scratch_types=(
            [
                pltpu.VMEM((batch, s_per_w), jnp.int32),
                pltpu.VMEM((s_per_w // chunk, chunk), jnp.int32),
            ]
            + [pltpu.VMEM((chunk, d_model), jnp.float32)] * nbuf
            + [pltpu.VMEM((chunk,), jnp.int32)]
            + [pltpu.SemaphoreType.DMA] * (2 * nbuf + 1)
        ),
    )
    def k(table_hbm, tok_hbm, out_hbm, tok_v, pidx_v, *scr):
        bufs = scr[:nbuf]
        fidx_v = scr[nbuf]
        rsems = scr[nbuf + 1:2 * nbuf + 1]
        wsems = scr[2 * nbuf + 1:3 * nbuf + 1]
        tsem = scr[3 * nbuf + 1]
        buf0 = bufs[0]
        wid = lax.axis_index("s") * _NC + lax.axis_index("c")
        s0 = wid * s_per_w

        for i in range(s_per_w // _LANES):
            pos = (s0 + (i * _LANES + 1)) + lax.iota(jnp.int32, 16)
            pidx_v[(i * _LANES) // chunk,
                   pl.ds((i * _LANES) % chunk, _LANES)] = pos

        def read_chunk(c):
            return pltpu.async_copy(
                table_hbm.at[pidx_v.at[c]], bufs[c % nbuf], rsems[c % nbuf])

        reads = [None] * n_chunks
        for c in range(min(nbuf, n_chunks)):
            reads[c] = read_chunk(c)
        tok_copies = [
            pltpu.async_copy(tok_hbm.at[b, pl.ds(s0, s_per_w)],
                             tok_v.at[b], tsem)
            for b in range(batch)
        ]

        writes = [[None] * batch for _ in range(n_chunks)]
        for c in range(n_chunks):
            reads[c].wait()
            if c == n_chunks - 2 and n_chunks > nbuf:
                for w in writes[0]:
                    w.wait()
                reads[n_chunks - 1] = read_chunk(n_chunks - 1)
            for b in range(batch):
                writes[c][b] = pltpu.async_copy(
                    bufs[c % nbuf],
                    out_hbm.at[pl.ds(b * seq_len + s0 + c * chunk, chunk)],
                    wsems[c % nbuf])
        for c in range(1 if n_chunks > nbuf else 0, n_chunks):
            for w in writes[c]:
                w.wait()
        for t in tok_copies:
            t.wait()

        def any_zero_scalar(vreg):
            anyz = None
            for j in range(_LANES):
                lz = vreg[j] == 0
                anyz = lz if anyz is None else (anyz | lz)
            return anyz

        def fix_span(cb, carry):
            c = cb // batch
            b = cb % batch
            t1 = tok_v[b, pl.ds(c * chunk, _LANES)]
            t2 = tok_v[b, pl.ds(c * chunk + _LANES, _LANES)]

            @pl.when(any_zero_scalar(jnp.minimum(t1, t2)))
            def _fix():
                for i in range(chunk // _LANES):
                    tok = tok_v[b, pl.ds(c * chunk + i * _LANES, _LANES)]
                    pos = (s0 + (c * chunk + i * _LANES + 1)
                           ) + lax.iota(jnp.int32, 16)
                    fidx_v[pl.ds(i * _LANES, _LANES)] = jnp.where(
                        tok == 0, 0, pos)
                pltpu.async_copy(
                    table_hbm.at[fidx_v], buf0, rsems[0]).wait()
                pltpu.async_copy(
                    buf0,
                    out_hbm.at[pl.ds(b * seq_len + s0 + c * chunk, chunk)],
                    wsems[0]).wait()

            return carry

        lax.fori_loop(0, n_chunks * batch, fix_span, 0)

    return k


def kernel(inputs, pos_table):
    batch, seq_len = inputs.shape
    d_model = pos_table.shape[1]
    tok = inputs.astype(jnp.int32)
    k = _build_sc_posenc(batch, seq_len, d_model)
    out = k(pos_table, tok)
    return out.reshape(batch, seq_len, d_model)
